# Initial kernel scaffold; baseline (speedup 1.0000x reference)
import jax
import jax.numpy as jnp
from jax.experimental import pallas as pl


def kernel(ent_id, adj_entity_list, A, context_ent_embed, ent_embed, entity_gcn_weight, gate_entity, v_ent):
    raise NotImplementedError("write your pallas kernel here")


# speedup vs baseline: 4.1158x; 4.1158x over previous
import functools

import jax
import jax.numpy as jnp
from jax import lax
from jax.experimental import pallas as pl
from jax.experimental.pallas import tpu as pltpu
from jax.experimental.pallas import tpu_sc as plsc


def _sc_gather(table1, table2, idx1, idx2):
    R1, = idx1.shape
    R2, = idx2.shape
    D = table1.shape[1]
    info = plsc.get_sparse_core_info()
    NW = info.num_cores * info.num_subcores
    r1 = R1 // NW
    r2 = R2 // NW
    CH = 80
    n_ch = r1 // CH
    assert r1 % CH == 0 and r1 % 8 == 0 and r2 % 8 == 0

    mesh = plsc.VectorSubcoreMesh(core_axis_name="c", subcore_axis_name="s")

    @functools.partial(
        pl.kernel,
        mesh=mesh,
        out_type=[
            jax.ShapeDtypeStruct((R1, D), jnp.float32),
            jax.ShapeDtypeStruct((R2, D), jnp.float32),
        ],
        scratch_types=[
            pltpu.VMEM((CH,), jnp.int32),
            pltpu.VMEM((CH, D), jnp.float32),
            pltpu.VMEM((r2,), jnp.int32),
            pltpu.VMEM((r2, D), jnp.float32),
            pltpu.SemaphoreType.DMA,
        ],
    )
    def k(t1, t2, i1, i2, out1, out2, idx_v, rows_v, idx2_v, rows2_v, sem):
        wid = lax.axis_index("s") * info.num_cores + lax.axis_index("c")
        base1 = wid * r1

        def body(c, carry):
            off = base1 + c * CH
            pltpu.sync_copy(i1.at[pl.ds(off, CH)], idx_v)
            pltpu.async_copy(t1.at[idx_v], rows_v, sem).wait()
            pltpu.sync_copy(rows_v, out1.at[pl.ds(off, CH)])
            return carry

        lax.fori_loop(0, n_ch, body, 0)

        base2 = wid * r2
        pltpu.sync_copy(i2.at[pl.ds(base2, r2)], idx2_v)
        pltpu.async_copy(t2.at[idx2_v], rows2_v, sem).wait()
        pltpu.sync_copy(rows2_v, out2.at[pl.ds(base2, r2)])

    return k(table1, table2, idx1, idx2)


def _tc_body(N, BB, a_ref, h_ref, o_ref, w_ref, gate_ref, v_ref, out_ref,
             sup_ref, gcn_ref):
    for b in range(BB):
        sup_ref[pl.ds(b * N, N), :] = jnp.dot(
            a_ref[b], h_ref[pl.ds(b * N, N), :],
            preferred_element_type=jnp.float32)
    gcn_ref[...] = jax.nn.relu(
        jnp.dot(sup_ref[...], w_ref[...], preferred_element_type=jnp.float32))
    gate = gate_ref[...]
    v = v_ref[...]
    for b in range(BB):
        gcn = gcn_ref[pl.ds(b * N, N), :]
        o_b = o_ref[pl.ds(b, 1), :]
        tmp = jax.nn.relu(gcn * o_b)
        logits = jnp.sum(tmp * v, axis=1, keepdims=True)
        m = jnp.max(logits)
        e = jnp.exp(logits - m)
        alpha = e / jnp.sum(e)
        sg = jnp.sum(alpha * gcn, axis=0, keepdims=True)
        out_ref[pl.ds(b, 1), :] = gate * o_b + (1.0 - gate) * sg


def _tc_compute(A, H2d, o, W, gate, v, N, BB, interpret=False):
    B = A.shape[0]
    D = W.shape[0]
    grid = (B // BB,)
    return pl.pallas_call(
        functools.partial(_tc_body, N, BB),
        grid=grid,
        in_specs=[
            pl.BlockSpec((BB, N, N), lambda i: (i, 0, 0)),
            pl.BlockSpec((BB * N, D), lambda i: (i, 0)),
            pl.BlockSpec((BB, D), lambda i: (i, 0)),
            pl.BlockSpec((D, D), lambda i: (0, 0)),
            pl.BlockSpec((1, D), lambda i: (0, 0)),
            pl.BlockSpec((1, D), lambda i: (0, 0)),
        ],
        out_specs=pl.BlockSpec((BB, D), lambda i: (i, 0)),
        out_shape=jax.ShapeDtypeStruct((B, D), jnp.float32),
        scratch_shapes=[
            pltpu.VMEM((BB * N, D), jnp.float32),
            pltpu.VMEM((BB * N, D), jnp.float32),
        ],
        interpret=interpret,
    )(A, H2d, o, W, gate, v)


def kernel(ent_id, adj_entity_list, A, context_ent_embed, ent_embed,
           entity_gcn_weight, gate_entity, v_ent):
    B, C = adj_entity_list.shape
    N = C + 1
    D = context_ent_embed.shape[1]
    idx_all = jnp.concatenate(
        [ent_id[:, None], adj_entity_list], axis=1).reshape(B * N)
    H2d, o = _sc_gather(context_ent_embed, ent_embed,
                        idx_all.astype(jnp.int32), ent_id.astype(jnp.int32))
    gate = gate_entity.reshape(1, D)
    v = v_ent.reshape(1, D)
    return _tc_compute(A, H2d, o, entity_gcn_weight, gate, v, N, BB=8)
